# baseline (speedup 1.0000x reference)
import functools

import jax
import jax.numpy as jnp
from jax import lax
from jax.experimental import pallas as pl
from jax.experimental.pallas import tpu as pltpu
from jax.experimental.pallas import tpu_sc as plsc

_N = 10000
_NP = 10240
_E = 320000
_D = 128
_H1 = 64
_H2 = 32
_W = 32

_NC = 2
_NS = 16
_NW = _NC * _NS
_CHUNK = 128
_NCHUNK = 80
_EPT = _NCHUNK * _CHUNK
_EP = _NW * _EPT
_RPT = _NP // _NS

_mesh = plsc.VectorSubcoreMesh(core_axis_name="c", subcore_axis_name="s")


@functools.partial(
    pl.kernel,
    out_type=jax.ShapeDtypeStruct((_NW, _NP), jnp.float32),
    mesh=_mesh,
    scratch_types=[
        pltpu.VMEM((_EPT,), jnp.int32),
        pltpu.VMEM((_NP,), jnp.float32),
    ],
    compiler_params=pltpu.CompilerParams(needs_layout_passes=False),
)
def _deg_kernel(dst_hbm, out_hbm, dst_v, deg_v):
    wid = lax.axis_index("c") * _NS + lax.axis_index("s")
    pltpu.sync_copy(dst_hbm.at[wid], dst_v)

    def zero_body(t, carry):
        deg_v[pl.ds(t * 16, 16)] = jnp.zeros((16,), jnp.float32)
        return carry

    lax.fori_loop(0, _NP // 16, zero_body, 0)
    ones = jnp.ones((16,), jnp.float32)

    def scat_body(t, carry):
        idx = dst_v[pl.ds(t * 16, 16)]
        plsc.addupdate_scatter(deg_v, [idx], ones)
        return carry

    lax.fori_loop(0, _EPT // 16, scat_body, 0)
    pltpu.sync_copy(deg_v, out_hbm.at[wid])


_NBUF = 4
_DBL = 4
_NSTEP = _NCHUNK // _DBL


def _make_agg(PH):
    @functools.partial(
        pl.kernel,
        out_type=jax.ShapeDtypeStruct((_NC, PH, _NP, _W), jnp.float32),
        mesh=_mesh,
        scratch_types=(
            [pltpu.VMEM((_NSTEP, _DBL * _CHUNK), jnp.int32)] * 2
            + [pltpu.VMEM((_DBL * _CHUNK, _W), jnp.float32)] * _NBUF
            + [pltpu.VMEM((_CHUNK, _W), jnp.float32)] * 2
            + [pltpu.VMEM_SHARED((_NP, _W), jnp.float32)] * 2
            + [pltpu.SemaphoreType.DMA] * (2 * _NBUF)
        ),
        compiler_params=pltpu.CompilerParams(use_tc_tiling_on_sc=False),
    )
    def _agg(src_hbm, dst_hbm, g_hbm, out_hbm, src_v, dst_v, *rest):
        rows = rest[:_NBUF]
        stage_v = rest[_NBUF]
        zero_v = rest[_NBUF + 1]
        acc_sh = rest[_NBUF + 2]
        g_sh = rest[_NBUF + 3]
        gsem = rest[_NBUF + 4:2 * _NBUF + 4]
        ssem = rest[2 * _NBUF + 4:]
        cid = lax.axis_index("c")
        sid = lax.axis_index("s")
        wid = cid * _NS + sid
        base = sid * _RPT

        pltpu.sync_copy(src_hbm.at[wid], src_v)
        pltpu.sync_copy(dst_hbm.at[wid], dst_v)

        def zero_body(t, carry):
            zero_v[t // 2, pl.ds((t % 2) * 16, 16)] = (
                jnp.zeros((16,), jnp.float32))
            return carry

        lax.fori_loop(0, _CHUNK * _W // 16, zero_body, 0)

        def g_cp(j, k):
            return pltpu.make_async_copy(g_sh.at[src_v.at[j]], rows[k],
                                         gsem[k])

        def s_start(j, k):
            return pltpu.async_copy(rows[k], acc_sh.at[dst_v.at[j]], ssem[k],
                                    add=True)

        for ph in range(PH):
            for k0 in range(0, _RPT // _CHUNK, _NBUF):
                ks = range(k0, min(k0 + _NBUF, _RPT // _CHUNK))
                for k in ks:
                    pltpu.async_copy(
                        g_hbm.at[ph, pl.ds(base + k * _CHUNK, _CHUNK)],
                        rows[k - k0].at[pl.ds(0, _CHUNK)], gsem[k - k0])
                for k in ks:
                    pltpu.make_async_copy(
                        g_hbm.at[ph, pl.ds(base + k * _CHUNK, _CHUNK)],
                        rows[k - k0].at[pl.ds(0, _CHUNK)], gsem[k - k0]).wait()
                    pltpu.sync_copy(rows[k - k0].at[pl.ds(0, _CHUNK)],
                                    g_sh.at[pl.ds(base + k * _CHUNK, _CHUNK)])
                    pltpu.sync_copy(
                        zero_v, acc_sh.at[pl.ds(base + k * _CHUNK, _CHUNK)])

            plsc.subcore_barrier()

            for k in range(_NBUF):
                g_cp(k, k).start()

            def body(jj, carry):
                j = jj * _NBUF
                scats = []
                for k in range(_NBUF):
                    g_cp(j + k, k).wait()
                    scats.append(s_start(j + k, k))
                for k in range(_NBUF):
                    scats[k].wait()
                    g_cp(j + _NBUF + k, k).start()
                return carry

            lax.fori_loop(0, _NSTEP // _NBUF - 1, body, 0)
            j = _NSTEP - _NBUF
            scats = []
            for k in range(_NBUF):
                g_cp(j + k, k).wait()
                scats.append(s_start(j + k, k))
            for k in range(_NBUF):
                scats[k].wait()
            plsc.subcore_barrier()

            for k0 in range(0, _RPT // _CHUNK, _NBUF):
                ks = range(k0, min(k0 + _NBUF, _RPT // _CHUNK))
                outs = []
                for k in ks:
                    pltpu.sync_copy(
                        acc_sh.at[pl.ds(base + k * _CHUNK, _CHUNK)],
                        rows[k - k0].at[pl.ds(0, _CHUNK)])
                    outs.append(pltpu.async_copy(
                        rows[k - k0].at[pl.ds(0, _CHUNK)],
                        out_hbm.at[cid, ph, pl.ds(base + k * _CHUNK, _CHUNK)],
                        gsem[k - k0]))
                for o in outs:
                    o.wait()

    return _agg


_agg64 = _make_agg(2)
_agg32 = _make_agg(1)


_BLK = 2048


def _dis_from(degp_blk):
    deg = jnp.sum(degp_blk, axis=0)[:, None]
    return jnp.where(deg > 0, lax.rsqrt(jnp.maximum(deg, 1.0)), 0.0)


def _tc1_body(x_ref, w1_ref, degp_ref, g_ref):
    dis = _dis_from(degp_ref[...])
    h = jnp.dot(x_ref[...], w1_ref[...], preferred_element_type=jnp.float32)
    g = h * dis
    g_ref[0] = g[:, :_W]
    g_ref[1] = g[:, _W:]


def _tc1(x_pad, w1, degp):
    return pl.pallas_call(
        _tc1_body,
        grid=(_NP // _BLK,),
        in_specs=[
            pl.BlockSpec((_BLK, _D), lambda i: (i, 0)),
            pl.BlockSpec((_D, _H1), lambda i: (0, 0)),
            pl.BlockSpec((_NW, _BLK), lambda i: (0, i)),
        ],
        out_specs=pl.BlockSpec((2, _BLK, _W), lambda i: (0, i, 0)),
        out_shape=jax.ShapeDtypeStruct((2, _NP, _W), jnp.float32),
    )(x_pad, w1, degp)


def _tc2_body(acc_ref, w2_ref, degp_ref, g2_ref):
    dis = _dis_from(degp_ref[...])
    a = jnp.concatenate(
        [acc_ref[0, 0] + acc_ref[1, 0], acc_ref[0, 1] + acc_ref[1, 1]],
        axis=-1)
    h1 = jnp.maximum(a * dis, 0.0)
    h2 = jnp.dot(h1, w2_ref[...], preferred_element_type=jnp.float32)
    g2_ref[0] = h2 * dis


def _tc2(acc1, w2, degp):
    return pl.pallas_call(
        _tc2_body,
        grid=(_NP // _BLK,),
        in_specs=[
            pl.BlockSpec((_NC, 2, _BLK, _W), lambda i: (0, 0, i, 0)),
            pl.BlockSpec((_H1, _H2), lambda i: (0, 0)),
            pl.BlockSpec((_NW, _BLK), lambda i: (0, i)),
        ],
        out_specs=pl.BlockSpec((1, _BLK, _W), lambda i: (0, i, 0)),
        out_shape=jax.ShapeDtypeStruct((1, _NP, _W), jnp.float32),
    )(acc1, w2, degp)


def _tc3_body(acc_ref, degp_ref, out_ref):
    dis = _dis_from(degp_ref[...])
    logits = (acc_ref[0, 0] + acc_ref[1, 0]) * dis
    m = jnp.max(logits, axis=-1, keepdims=True)
    e = jnp.exp(logits - m)
    out_ref[...] = e / jnp.sum(e, axis=-1, keepdims=True)


def _tc3(acc2, degp):
    return pl.pallas_call(
        _tc3_body,
        grid=(_NP // _BLK,),
        in_specs=[
            pl.BlockSpec((_NC, 1, _BLK, _W), lambda i: (0, 0, i, 0)),
            pl.BlockSpec((_NW, _BLK), lambda i: (0, i)),
        ],
        out_specs=pl.BlockSpec((_BLK, _H2), lambda i: (i, 0)),
        out_shape=jax.ShapeDtypeStruct((_N, _H2), jnp.float32),
    )(acc2, degp)


def kernel(x, edge_index, W1, W2):
    x_pad = jnp.pad(x, ((0, _NP - _N), (0, 0)))
    pad = jnp.full((_EP - _E,), _N, jnp.int32)
    src = jnp.concatenate([edge_index[0], pad])
    dst = jnp.concatenate([edge_index[1], pad])
    src_r = src.reshape(_NW, _NSTEP, _DBL * _CHUNK)
    dst_r = dst.reshape(_NW, _NSTEP, _DBL * _CHUNK)
    dst_f = dst.reshape(_NW, _EPT)

    degp = _deg_kernel(dst_f)
    g = _tc1(x_pad, W1, degp)
    acc1 = _agg64(src_r, dst_r, g)
    g2 = _tc2(acc1, W2, degp)
    acc2 = _agg32(src_r, dst_r, g2)
    return _tc3(acc2, degp)


# speedup vs baseline: 1.0376x; 1.0376x over previous
import functools

import jax
import jax.numpy as jnp
from jax import lax
from jax.experimental import pallas as pl
from jax.experimental.pallas import tpu as pltpu
from jax.experimental.pallas import tpu_sc as plsc

_N = 10000
_NP = 10240
_E = 320000
_D = 128
_H1 = 64
_H2 = 32
_W = 32

_NC = 2
_NS = 16
_NW = _NC * _NS
_CHUNK = 128
_NCHUNK = 80
_EPT = _NCHUNK * _CHUNK
_EP = _NW * _EPT
_RPT = _NP // _NS

_mesh = plsc.VectorSubcoreMesh(core_axis_name="c", subcore_axis_name="s")


@functools.partial(
    pl.kernel,
    out_type=jax.ShapeDtypeStruct((_NW, _NP), jnp.float32),
    mesh=_mesh,
    scratch_types=[
        pltpu.VMEM((_EPT,), jnp.int32),
        pltpu.VMEM((_NP,), jnp.float32),
    ],
    compiler_params=pltpu.CompilerParams(needs_layout_passes=False),
)
def _deg_kernel(dst_hbm, out_hbm, dst_v, deg_v):
    wid = lax.axis_index("c") * _NS + lax.axis_index("s")
    pltpu.sync_copy(dst_hbm.at[wid], dst_v)

    def zero_body(t, carry):
        deg_v[pl.ds(t * 16, 16)] = jnp.zeros((16,), jnp.float32)
        return carry

    lax.fori_loop(0, _NP // 16, zero_body, 0)
    ones = jnp.ones((16,), jnp.float32)

    def scat_body(t, carry):
        idx = dst_v[pl.ds(t * 16, 16)]
        plsc.addupdate_scatter(deg_v, [idx], ones)
        return carry

    lax.fori_loop(0, _EPT // 16, scat_body, 0)
    pltpu.sync_copy(deg_v, out_hbm.at[wid])


_NBUF = 8
_DBL = 2
_NSTEP = _NCHUNK // _DBL


def _make_agg(PH):
    @functools.partial(
        pl.kernel,
        out_type=jax.ShapeDtypeStruct((_NC, PH, _NP, _W), jnp.float32),
        mesh=_mesh,
        scratch_types=(
            [pltpu.VMEM((_NSTEP, _DBL * _CHUNK), jnp.int32)] * 2
            + [pltpu.VMEM((_DBL * _CHUNK, _W), jnp.float32)] * _NBUF
            + [pltpu.VMEM((_CHUNK, _W), jnp.float32)] * 2
            + [pltpu.VMEM_SHARED((_NP, _W), jnp.float32)] * 2
            + [pltpu.SemaphoreType.DMA] * (2 * _NBUF)
        ),
        compiler_params=pltpu.CompilerParams(use_tc_tiling_on_sc=False),
    )
    def _agg(src_hbm, dst_hbm, g_hbm, out_hbm, src_v, dst_v, *rest):
        rows = rest[:_NBUF]
        stage_v = rest[_NBUF]
        zero_v = rest[_NBUF + 1]
        acc_sh = rest[_NBUF + 2]
        g_sh = rest[_NBUF + 3]
        gsem = rest[_NBUF + 4:2 * _NBUF + 4]
        ssem = rest[2 * _NBUF + 4:]
        cid = lax.axis_index("c")
        sid = lax.axis_index("s")
        wid = cid * _NS + sid
        base = sid * _RPT

        pltpu.sync_copy(src_hbm.at[wid], src_v)
        pltpu.sync_copy(dst_hbm.at[wid], dst_v)

        def zero_body(t, carry):
            zero_v[t // 2, pl.ds((t % 2) * 16, 16)] = (
                jnp.zeros((16,), jnp.float32))
            return carry

        lax.fori_loop(0, _CHUNK * _W // 16, zero_body, 0)

        def g_cp(j, k):
            return pltpu.make_async_copy(g_sh.at[src_v.at[j]], rows[k],
                                         gsem[k])

        def s_start(j, k):
            return pltpu.async_copy(rows[k], acc_sh.at[dst_v.at[j]], ssem[k],
                                    add=True)

        for ph in range(PH):
            for k0 in range(0, _RPT // _CHUNK, _NBUF):
                ks = range(k0, min(k0 + _NBUF, _RPT // _CHUNK))
                for k in ks:
                    pltpu.async_copy(
                        g_hbm.at[ph, pl.ds(base + k * _CHUNK, _CHUNK)],
                        rows[k - k0].at[pl.ds(0, _CHUNK)], gsem[k - k0])
                for k in ks:
                    pltpu.make_async_copy(
                        g_hbm.at[ph, pl.ds(base + k * _CHUNK, _CHUNK)],
                        rows[k - k0].at[pl.ds(0, _CHUNK)], gsem[k - k0]).wait()
                    pltpu.sync_copy(rows[k - k0].at[pl.ds(0, _CHUNK)],
                                    g_sh.at[pl.ds(base + k * _CHUNK, _CHUNK)])
                    pltpu.sync_copy(
                        zero_v, acc_sh.at[pl.ds(base + k * _CHUNK, _CHUNK)])

            plsc.subcore_barrier()

            for k in range(_NBUF):
                g_cp(k, k).start()

            def body(jj, carry):
                j = jj * _NBUF
                scats = []
                for k in range(_NBUF):
                    g_cp(j + k, k).wait()
                    scats.append(s_start(j + k, k))
                for k in range(_NBUF):
                    scats[k].wait()
                    g_cp(j + _NBUF + k, k).start()
                return carry

            lax.fori_loop(0, _NSTEP // _NBUF - 1, body, 0)
            j = _NSTEP - _NBUF
            scats = []
            for k in range(_NBUF):
                g_cp(j + k, k).wait()
                scats.append(s_start(j + k, k))
            for k in range(_NBUF):
                scats[k].wait()
            plsc.subcore_barrier()

            for k0 in range(0, _RPT // _CHUNK, _NBUF):
                ks = range(k0, min(k0 + _NBUF, _RPT // _CHUNK))
                outs = []
                for k in ks:
                    pltpu.sync_copy(
                        acc_sh.at[pl.ds(base + k * _CHUNK, _CHUNK)],
                        rows[k - k0].at[pl.ds(0, _CHUNK)])
                    outs.append(pltpu.async_copy(
                        rows[k - k0].at[pl.ds(0, _CHUNK)],
                        out_hbm.at[cid, ph, pl.ds(base + k * _CHUNK, _CHUNK)],
                        gsem[k - k0]))
                for o in outs:
                    o.wait()

    return _agg


_agg64 = _make_agg(2)
_agg32 = _make_agg(1)


_BLK = 2048


def _dis_from(degp_blk):
    deg = jnp.sum(degp_blk, axis=0)[:, None]
    return jnp.where(deg > 0, lax.rsqrt(jnp.maximum(deg, 1.0)), 0.0)


def _tc1_body(x_ref, w1_ref, degp_ref, g_ref):
    dis = _dis_from(degp_ref[...])
    h = jnp.dot(x_ref[...], w1_ref[...], preferred_element_type=jnp.float32)
    g = h * dis
    g_ref[0] = g[:, :_W]
    g_ref[1] = g[:, _W:]


def _tc1(x_pad, w1, degp):
    return pl.pallas_call(
        _tc1_body,
        grid=(_NP // _BLK,),
        in_specs=[
            pl.BlockSpec((_BLK, _D), lambda i: (i, 0)),
            pl.BlockSpec((_D, _H1), lambda i: (0, 0)),
            pl.BlockSpec((_NW, _BLK), lambda i: (0, i)),
        ],
        out_specs=pl.BlockSpec((2, _BLK, _W), lambda i: (0, i, 0)),
        out_shape=jax.ShapeDtypeStruct((2, _NP, _W), jnp.float32),
    )(x_pad, w1, degp)


def _tc2_body(acc_ref, w2_ref, degp_ref, g2_ref):
    dis = _dis_from(degp_ref[...])
    a = jnp.concatenate(
        [acc_ref[0, 0] + acc_ref[1, 0], acc_ref[0, 1] + acc_ref[1, 1]],
        axis=-1)
    h1 = jnp.maximum(a * dis, 0.0)
    h2 = jnp.dot(h1, w2_ref[...], preferred_element_type=jnp.float32)
    g2_ref[0] = h2 * dis


def _tc2(acc1, w2, degp):
    return pl.pallas_call(
        _tc2_body,
        grid=(_NP // _BLK,),
        in_specs=[
            pl.BlockSpec((_NC, 2, _BLK, _W), lambda i: (0, 0, i, 0)),
            pl.BlockSpec((_H1, _H2), lambda i: (0, 0)),
            pl.BlockSpec((_NW, _BLK), lambda i: (0, i)),
        ],
        out_specs=pl.BlockSpec((1, _BLK, _W), lambda i: (0, i, 0)),
        out_shape=jax.ShapeDtypeStruct((1, _NP, _W), jnp.float32),
    )(acc1, w2, degp)


def _tc3_body(acc_ref, degp_ref, out_ref):
    dis = _dis_from(degp_ref[...])
    logits = (acc_ref[0, 0] + acc_ref[1, 0]) * dis
    m = jnp.max(logits, axis=-1, keepdims=True)
    e = jnp.exp(logits - m)
    out_ref[...] = e / jnp.sum(e, axis=-1, keepdims=True)


def _tc3(acc2, degp):
    return pl.pallas_call(
        _tc3_body,
        grid=(_NP // _BLK,),
        in_specs=[
            pl.BlockSpec((_NC, 1, _BLK, _W), lambda i: (0, 0, i, 0)),
            pl.BlockSpec((_NW, _BLK), lambda i: (0, i)),
        ],
        out_specs=pl.BlockSpec((_BLK, _H2), lambda i: (i, 0)),
        out_shape=jax.ShapeDtypeStruct((_N, _H2), jnp.float32),
    )(acc2, degp)


def kernel(x, edge_index, W1, W2):
    x_pad = jnp.pad(x, ((0, _NP - _N), (0, 0)))
    pad = jnp.full((_EP - _E,), _N, jnp.int32)
    src = jnp.concatenate([edge_index[0], pad])
    dst = jnp.concatenate([edge_index[1], pad])
    src_r = src.reshape(_NW, _NSTEP, _DBL * _CHUNK)
    dst_r = dst.reshape(_NW, _NSTEP, _DBL * _CHUNK)
    dst_f = dst.reshape(_NW, _EPT)

    degp = _deg_kernel(dst_f)
    g = _tc1(x_pad, W1, degp)
    acc1 = _agg64(src_r, dst_r, g)
    g2 = _tc2(acc1, W2, degp)
    acc2 = _agg32(src_r, dst_r, g2)
    return _tc3(acc2, degp)
